# baseline (speedup 1.0000x reference)
import functools
import jax
import jax.numpy as jnp
from jax import lax
from jax.experimental import pallas as pl
from jax.experimental.pallas import tpu as pltpu
from jax.experimental.pallas import tpu_sc as plsc

_D = 16
_B = 16384
_NW = 32
_BPW = _B // _NW
_G = 16
_NG = _BPW // _G

_mesh = plsc.VectorSubcoreMesh(core_axis_name="c", subcore_axis_name="s")


@functools.partial(
    pl.kernel,
    mesh=_mesh,
    compiler_params=pltpu.CompilerParams(
        needs_layout_passes=False, skip_device_barrier=True
    ),
    out_type=jax.ShapeDtypeStruct((_D, _B), jnp.float32),
    scratch_types=[
        pltpu.VMEM((_BPW,), jnp.int32),
        pltpu.VMEM((3 * _G * _D, 128), jnp.float32),
        pltpu.VMEM((_D, _BPW), jnp.float32),
        pltpu.SemaphoreType.DMA,
        pltpu.SemaphoreType.DMA,
        pltpu.SemaphoreType.DMA,
        pltpu.SemaphoreType.DMA,
    ],
)
def _lookup(idx_hbm, table_t_hbm, out_hbm, idx_v, tiles, colbuf, sem0, sem1,
            sem2, sem3):
    wid = lax.axis_index("s") * 2 + lax.axis_index("c")
    base = wid * _BPW
    pltpu.sync_copy(idx_hbm.at[pl.ds(base, _BPW)], idx_v)
    rows = lax.iota(jnp.int32, 16)
    sems = [sem0, sem1, sem2]

    def fire(g, b):
        vec = idx_v[pl.ds(g * _G, _G)]
        for l in range(_G):
            v = vec[l]
            cal = pl.multiple_of((v >> 7) * 128, 128)
            pltpu.async_copy(
                table_t_hbm.at[:, pl.ds(cal, 128)],
                tiles.at[pl.ds((b * _G + l) * _D, _D), :],
                sems[b],
            )

    def drain(b):
        for l in range(_G):
            pltpu.make_async_copy(
                table_t_hbm.at[:, pl.ds(0, 128)],
                tiles.at[pl.ds((b * _G + l) * _D, _D), :],
                sems[b],
            ).wait()

    def extract(g, b):
        vec = idx_v[pl.ds(g * _G, _G)]
        for l in range(_G):
            v = vec[l]
            w = jnp.full((16,), v & 127, jnp.int32)
            emb = plsc.load_gather(tiles, [(b * _G + l) * _D + rows, w])
            j = jnp.full((16,), g * _G + l, jnp.int32)
            plsc.store_scatter(colbuf, [rows, j], emb)

    def body(k, carry):
        for j in range(3):
            g = k * 3 + j

            @pl.when(g + 2 < _NG)
            def _(g=g, j=j):
                fire(g + 2, (j + 2) % 3)

            @pl.when(g < _NG)
            def _(g=g, j=j):
                drain(j)
                extract(g, j)

                @pl.when(lax.rem(g, 8) == 7)
                def _(g=g):
                    q = (g // 8) * 128
                    pltpu.async_copy(
                        colbuf.at[:, pl.ds(q, 128)],
                        out_hbm.at[:, pl.ds(base + q, 128)],
                        sem3,
                    )
        return carry

    fire(0, 0)
    fire(1, 1)
    lax.fori_loop(0, (_NG + 2) // 3, body, 0)
    for q in range(_NG // 8):
        pltpu.make_async_copy(
            table_t_hbm.at[:, pl.ds(0, 128)],
            colbuf.at[:, pl.ds(q * 128, 128)],
            sem3,
        ).wait()


def kernel(value, table):
    table_t = jnp.swapaxes(table, 0, 1)
    out_t = _lookup(value, table_t)
    return jnp.swapaxes(out_t, 0, 1)


# speedup vs baseline: 1.0132x; 1.0132x over previous
import functools
import jax
import jax.numpy as jnp
from jax import lax
from jax.experimental import pallas as pl
from jax.experimental.pallas import tpu as pltpu
from jax.experimental.pallas import tpu_sc as plsc

_D = 16
_B = 16384
_NW = 32
_BPW = _B // _NW
_G = 16
_NG = _BPW // _G

_mesh = plsc.VectorSubcoreMesh(core_axis_name="c", subcore_axis_name="s")


@functools.partial(
    pl.kernel,
    mesh=_mesh,
    compiler_params=pltpu.CompilerParams(needs_layout_passes=False),
    out_type=jax.ShapeDtypeStruct((_D, _B), jnp.float32),
    scratch_types=[
        pltpu.VMEM((_BPW,), jnp.int32),
        pltpu.VMEM((3 * _G * _D, 128), jnp.float32),
        pltpu.VMEM((_D, _BPW), jnp.float32),
        pltpu.SemaphoreType.DMA,
        pltpu.SemaphoreType.DMA,
        pltpu.SemaphoreType.DMA,
        pltpu.SemaphoreType.DMA,
    ],
)
def _lookup(idx_hbm, table_t_hbm, out_hbm, idx_v, tiles, colbuf, sem0, sem1,
            sem2, sem3):
    wid = lax.axis_index("s") * 2 + lax.axis_index("c")
    base = wid * _BPW
    pltpu.sync_copy(idx_hbm.at[pl.ds(base, _BPW)], idx_v)
    rows = lax.iota(jnp.int32, 16)
    sems = [sem0, sem1, sem2]

    def fire(g, b):
        vec = idx_v[pl.ds(g * _G, _G)]
        for l in range(_G):
            v = vec[l]
            cal = pl.multiple_of((v >> 7) * 128, 128)
            pltpu.async_copy(
                table_t_hbm.at[:, pl.ds(cal, 128)],
                tiles.at[pl.ds((b * _G + l) * _D, _D), :],
                sems[b],
            )

    def drain(b):
        for l in range(_G):
            pltpu.make_async_copy(
                table_t_hbm.at[:, pl.ds(0, 128)],
                tiles.at[pl.ds((b * _G + l) * _D, _D), :],
                sems[b],
            ).wait()

    def extract(g, b):
        vec = idx_v[pl.ds(g * _G, _G)]
        for l in range(_G):
            v = vec[l]
            w = jnp.full((16,), v & 127, jnp.int32)
            emb = plsc.load_gather(tiles, [(b * _G + l) * _D + rows, w])
            j = jnp.full((16,), g * _G + l, jnp.int32)
            plsc.store_scatter(colbuf, [rows, j], emb)

    def body(k, carry):
        for j in range(3):
            g = k * 3 + j

            @pl.when(g + 2 < _NG)
            def _(g=g, j=j):
                fire(g + 2, (j + 2) % 3)

            @pl.when(g < _NG)
            def _(g=g, j=j):
                drain(j)
                extract(g, j)

                @pl.when(lax.rem(g, 8) == 7)
                def _(g=g):
                    q = (g // 8) * 128
                    pltpu.async_copy(
                        colbuf.at[:, pl.ds(q, 128)],
                        out_hbm.at[:, pl.ds(base + q, 128)],
                        sem3,
                    )
        return carry

    fire(0, 0)
    fire(1, 1)
    lax.fori_loop(0, (_NG + 2) // 3, body, 0)
    for q in range(_NG // 8):
        pltpu.make_async_copy(
            table_t_hbm.at[:, pl.ds(0, 128)],
            colbuf.at[:, pl.ds(q * 128, 128)],
            sem3,
        ).wait()


def kernel(value, table):
    table_t = jnp.swapaxes(table, 0, 1)
    out_t = _lookup(value, table_t)
    return jnp.swapaxes(out_t, 0, 1)
